# baseline (speedup 1.0000x reference)
import functools

import jax
import jax.numpy as jnp
from jax import lax
from jax.experimental import pallas as pl
from jax.experimental.pallas import tpu as pltpu
from jax.experimental.pallas import tpu_sc as plsc

_CHUNK = 128


def _make_sc_gather(nw, nc, nchunks, k):
    half = nchunks // 2
    mesh = plsc.VectorSubcoreMesh(core_axis_name="c", subcore_axis_name="s")

    @functools.partial(
        pl.kernel,
        mesh=mesh,
        compiler_params=pltpu.CompilerParams(use_tc_tiling_on_sc=False),
        out_type=jax.ShapeDtypeStruct((nw, nchunks, _CHUNK, k), jnp.float32),
        scratch_types=[
            pltpu.VMEM((nchunks, _CHUNK), jnp.int32),
            pltpu.VMEM((half, _CHUNK, k), jnp.float32),
            pltpu.SemaphoreType.DMA,
        ],
    )
    def sc_gather(seq_hbm, table_hbm, out_hbm, idx_v, rows_v, sem):
        w = lax.axis_index("s") * nc + lax.axis_index("c")
        pltpu.sync_copy(seq_hbm.at[w], idx_v)

        def do_half(h, carry):
            base = h * half

            def fire(j, c2):
                pltpu.make_async_copy(
                    table_hbm.at[idx_v.at[base + j]], rows_v.at[j], sem
                ).start()
                return c2

            lax.fori_loop(0, half, fire, 0)

            def drain(j, c2):
                pltpu.make_async_copy(
                    table_hbm.at[idx_v.at[base + j]], rows_v.at[j], sem
                ).wait()
                return c2

            lax.fori_loop(0, half, drain, 0)
            pltpu.sync_copy(rows_v, out_hbm.at[w].at[pl.ds(base, half)])
            return carry

        lax.fori_loop(0, 2, do_half, 0)

    return sc_gather


def _tc_body(g_ref, w_ref, o_ref):
    s = jax.nn.sigmoid(g_ref[...])
    x = jnp.dot(s, w_ref[...], preferred_element_type=jnp.float32)
    z = -2.0 * x
    o_ref[...] = -(jnp.maximum(z, 0.0) + jnp.log1p(jnp.exp(-jnp.abs(z))))


def _tc_call(g2, wp, block_rows):
    n, kk = g2.shape
    return pl.pallas_call(
        _tc_body,
        grid=(n // block_rows,),
        in_specs=[
            pl.BlockSpec((block_rows, kk), lambda i: (i, 0)),
            pl.BlockSpec(wp.shape, lambda i: (0, 0)),
        ],
        out_specs=pl.BlockSpec((block_rows, wp.shape[1]), lambda i: (i, 0)),
        out_shape=jax.ShapeDtypeStruct((n, wp.shape[1]), jnp.float32),
    )(g2, wp)


def kernel(problem_seq, skill_offset, skill_slope, membership_logits, decoder):
    b, t = problem_seq.shape
    n_problems, k = membership_logits.shape
    n_states = decoder.shape[0]

    info = plsc.get_sparse_core_info()
    nc, ns = info.num_cores, info.num_subcores
    nw = nc * ns
    n = b * t
    nchunks = n // (nw * _CHUNK)

    state_logits = skill_offset[None, :] + skill_slope[None, :] * decoder
    st_t = state_logits.T
    w = jnp.stack([-st_t, st_t], axis=-1).reshape(k, 2 * n_states)
    w8 = jnp.concatenate([w, jnp.zeros_like(w)], axis=0)
    wp = jnp.kron(jnp.eye(4, dtype=w.dtype), w8)

    table8 = jnp.concatenate(
        [membership_logits, jnp.zeros_like(membership_logits)], axis=1)

    seq = problem_seq.reshape(nw, nchunks, _CHUNK).astype(jnp.int32)
    gathered = _make_sc_gather(nw, nc, nchunks, 2 * k)(seq, table8)
    g2 = gathered.reshape(n // 4, 8 * k)
    out = _tc_call(g2, wp, block_rows=8192)
    return out.reshape(b, t, n_states, 2)


# speedup vs baseline: 16.5270x; 16.5270x over previous
import functools

import jax
import jax.numpy as jnp
from jax import lax
from jax.experimental import pallas as pl
from jax.experimental.pallas import tpu as pltpu
from jax.experimental.pallas import tpu_sc as plsc

_CHUNK = 128
_PPAD = 1 << 20


def _make_sc_gather(nw, nc, nchunks, k):
    half = nchunks // 2
    mesh = plsc.VectorSubcoreMesh(core_axis_name="c", subcore_axis_name="s")

    @functools.partial(
        pl.kernel,
        mesh=mesh,
        compiler_params=pltpu.CompilerParams(use_tc_tiling_on_sc=False),
        out_type=jax.ShapeDtypeStruct((nw, nchunks, _CHUNK, k), jnp.float32),
        scratch_types=[
            pltpu.VMEM((nchunks, _CHUNK), jnp.int32),
            pltpu.VMEM((half, _CHUNK, k), jnp.float32),
            pltpu.SemaphoreType.DMA,
        ],
    )
    def sc_gather(seq_hbm, table_hbm, out_hbm, idx_v, rows_v, sem):
        w = lax.axis_index("s") * nc + lax.axis_index("c")
        pltpu.sync_copy(seq_hbm.at[w], idx_v)

        def do_half(h, carry):
            base = h * half

            def fire(j, c2):
                pltpu.make_async_copy(
                    table_hbm.at[idx_v.at[base + j]], rows_v.at[j], sem
                ).start()
                return c2

            lax.fori_loop(0, half, fire, 0)

            def drain(j, c2):
                pltpu.make_async_copy(
                    table_hbm.at[idx_v.at[base + j]], rows_v.at[j], sem
                ).wait()
                return c2

            lax.fori_loop(0, half, drain, 0)
            pltpu.sync_copy(rows_v, out_hbm.at[w].at[pl.ds(base, half)])
            return carry

        lax.fori_loop(0, 2, do_half, 0)

    return sc_gather


def _table_body(c0_ref, c1_ref, c2_ref, c3_ref, m_ref, o_ref):
    s = jnp.concatenate(
        [jax.nn.sigmoid(c0_ref[...]), jax.nn.sigmoid(c1_ref[...]),
         jax.nn.sigmoid(c2_ref[...]), jax.nn.sigmoid(c3_ref[...])],
        axis=1)
    for g in range(8):
        o_ref[:, g, :] = jnp.dot(s, m_ref[g],
                                 preferred_element_type=jnp.float32)


def _table_call(cols, mbig):
    blk = 512
    n = cols[0].shape[0]
    return pl.pallas_call(
        _table_body,
        grid=(n // blk,),
        in_specs=[pl.BlockSpec((blk, 128), lambda i: (i, 0))] * 4
        + [pl.BlockSpec(mbig.shape, lambda i: (0, 0, 0))],
        out_specs=pl.BlockSpec((blk, 8, 128), lambda i: (i, 0, 0)),
        out_shape=jax.ShapeDtypeStruct((n, 8, 128), jnp.float32),
    )(*cols, mbig)


def _tc_body(g_ref, w_ref, o0_ref, o1_ref):
    x = jnp.dot(g_ref[...], w_ref[...], preferred_element_type=jnp.float32)
    u = jnp.abs(x)
    lse = u + jnp.log1p(jnp.exp(-2.0 * u))
    o0_ref[...] = -x - lse
    o1_ref[...] = x - lse


def _tc_call(g2, wp):
    n = g2.shape[0]
    blk = 2048
    return pl.pallas_call(
        _tc_body,
        grid=(n // blk,),
        in_specs=[
            pl.BlockSpec((blk, 128), lambda i: (i, 0)),
            pl.BlockSpec(wp.shape, lambda i: (0, 0)),
        ],
        out_specs=[
            pl.BlockSpec((blk, 256), lambda i: (i, 0)),
            pl.BlockSpec((blk, 256), lambda i: (i, 0)),
        ],
        out_shape=[
            jax.ShapeDtypeStruct((n, 256), jnp.float32),
            jax.ShapeDtypeStruct((n, 256), jnp.float32),
        ],
    )(g2, wp)


def kernel(problem_seq, skill_offset, skill_slope, membership_logits, decoder):
    b, t = problem_seq.shape
    n_problems, k = membership_logits.shape
    n_states = decoder.shape[0]

    info = plsc.get_sparse_core_info()
    nc, ns = info.num_cores, info.num_subcores
    nw = nc * ns
    n = b * t
    nchunks = n // (nw * _CHUNK)

    state_logits = skill_offset[None, :] + skill_slope[None, :] * decoder
    st_t = state_logits.T
    w8 = jnp.concatenate([st_t, jnp.zeros_like(st_t)], axis=0)
    wp = jnp.kron(jnp.eye(16, dtype=w8.dtype), w8)

    mbig = jnp.zeros((8, 4 * 128, 128), dtype=jnp.float32)
    lane = jnp.arange(128)
    g_idx = lane // 16
    q_idx = lane % 16
    for c in range(4):
        mbig = mbig.at[g_idx, c * 128 + lane, 8 * q_idx + c].set(1.0)

    cols = [
        jnp.pad(membership_logits[:, c], (0, _PPAD - n_problems)).reshape(
            _PPAD // 128, 128)
        for c in range(k)
    ]
    table8 = _table_call(cols, mbig).reshape(_PPAD, 2 * k)

    seq = problem_seq.reshape(nw, nchunks, _CHUNK).astype(jnp.int32)
    gathered = _make_sc_gather(nw, nc, nchunks, 2 * k)(seq, table8)
    g2 = gathered.reshape(n // 16, 128)
    o0, o1 = _tc_call(g2, wp)

    one = 1.0 + 0.0 * skill_offset[0]

    def to_btS(o):
        ob = (o.reshape(b, t * n_states).T) * one
        return ob.reshape(t, n_states, b).transpose(2, 0, 1)

    o0r = to_btS(o0)
    o1r = to_btS(o1)
    return jnp.concatenate(
        [o0r[:, :, :, None], o1r[:, :, :, None]], axis=3) * one


# speedup vs baseline: 19.8611x; 1.2017x over previous
import functools

import jax
import jax.numpy as jnp
import numpy as np
from jax import lax
from jax.experimental import pallas as pl
from jax.experimental.pallas import tpu as pltpu
from jax.experimental.pallas import tpu_sc as plsc

_CHUNK = 128
_PPAD = 1 << 20


def _make_sc_gather(nw, nc, nchunks, k):
    half = nchunks // 2
    mesh = plsc.VectorSubcoreMesh(core_axis_name="c", subcore_axis_name="s")

    @functools.partial(
        pl.kernel,
        mesh=mesh,
        compiler_params=pltpu.CompilerParams(use_tc_tiling_on_sc=False),
        out_type=jax.ShapeDtypeStruct((nw, nchunks, _CHUNK, k), jnp.float32),
        scratch_types=[
            pltpu.VMEM((nchunks, _CHUNK), jnp.int32),
            pltpu.VMEM((half, _CHUNK, k), jnp.float32),
            pltpu.SemaphoreType.DMA,
        ],
    )
    def sc_gather(seq_hbm, table_hbm, out_hbm, idx_v, rows_v, sem):
        w = lax.axis_index("s") * nc + lax.axis_index("c")
        pltpu.sync_copy(seq_hbm.at[w], idx_v)

        def do_half(h, carry):
            base = h * half

            def fire(j, c2):
                pltpu.make_async_copy(
                    table_hbm.at[idx_v.at[base + j]], rows_v.at[j], sem
                ).start()
                return c2

            lax.fori_loop(0, half, fire, 0)

            def drain(j, c2):
                pltpu.make_async_copy(
                    table_hbm.at[idx_v.at[base + j]], rows_v.at[j], sem
                ).wait()
                return c2

            lax.fori_loop(0, half, drain, 0)
            pltpu.sync_copy(rows_v, out_hbm.at[w].at[pl.ds(base, half)])
            return carry

        lax.fori_loop(0, 2, do_half, 0)

    return sc_gather


def _table_body(c0_ref, c1_ref, c2_ref, c3_ref, m_ref, o_ref):
    s = jnp.concatenate(
        [jax.nn.sigmoid(c0_ref[...]), jax.nn.sigmoid(c1_ref[...]),
         jax.nn.sigmoid(c2_ref[...]), jax.nn.sigmoid(c3_ref[...])],
        axis=1)
    for g in range(8):
        o_ref[:, g, :] = jnp.dot(s, m_ref[g],
                                 preferred_element_type=jnp.float32)


def _table_call(cols, mbig):
    blk = 512
    n = cols[0].shape[0]
    return pl.pallas_call(
        _table_body,
        grid=(n // blk,),
        in_specs=[pl.BlockSpec((blk, 128), lambda i: (i, 0))] * 4
        + [pl.BlockSpec(mbig.shape, lambda i: (0, 0, 0))],
        out_specs=pl.BlockSpec((blk, 8, 128), lambda i: (i, 0, 0)),
        out_shape=jax.ShapeDtypeStruct((n, 8, 128), jnp.float32),
    )(*cols, mbig)


def _tc_body(g_ref, w_ref, o0_ref, o1_ref):
    x = jnp.dot(g_ref[...], w_ref[...], preferred_element_type=jnp.float32)
    u = jnp.abs(x)
    lse = u + jnp.log1p(jnp.exp(-2.0 * u))
    o0_ref[...] = -x - lse
    o1_ref[...] = x - lse


def _tc_call(g2, wp):
    n = g2.shape[0]
    blk = 2048
    return pl.pallas_call(
        _tc_body,
        grid=(n // blk,),
        in_specs=[
            pl.BlockSpec((blk, 128), lambda i: (i, 0)),
            pl.BlockSpec(wp.shape, lambda i: (0, 0)),
        ],
        out_specs=[
            pl.BlockSpec((blk, 256), lambda i: (i, 0)),
            pl.BlockSpec((blk, 256), lambda i: (i, 0)),
        ],
        out_shape=[
            jax.ShapeDtypeStruct((n, 256), jnp.float32),
            jax.ShapeDtypeStruct((n, 256), jnp.float32),
        ],
    )(g2, wp)


def kernel(problem_seq, skill_offset, skill_slope, membership_logits, decoder):
    b, t = problem_seq.shape
    n_problems, k = membership_logits.shape
    n_states = decoder.shape[0]

    info = plsc.get_sparse_core_info()
    nc, ns = info.num_cores, info.num_subcores
    nw = nc * ns
    n = b * t
    nchunks = n // (nw * _CHUNK)

    state_logits = skill_offset[None, :] + skill_slope[None, :] * decoder
    st_t = state_logits.T
    w8 = jnp.concatenate([st_t, jnp.zeros_like(st_t)], axis=0)
    wp = jnp.kron(jnp.eye(16, dtype=w8.dtype), w8)

    mbig_np = np.zeros((8, 4 * 128, 128), dtype=np.float32)
    for c in range(4):
        for l in range(128):
            mbig_np[l // 16, c * 128 + l, 8 * (l % 16) + c] = 1.0
    mbig = jnp.asarray(mbig_np)

    cols = [
        jnp.pad(membership_logits[:, c], (0, _PPAD - n_problems)).reshape(
            _PPAD // 128, 128)
        for c in range(k)
    ]
    table8 = _table_call(cols, mbig).reshape(_PPAD, 2 * k)

    seq = problem_seq.reshape(nw, nchunks, _CHUNK).astype(jnp.int32)
    gathered = _make_sc_gather(nw, nc, nchunks, 2 * k)(seq, table8)
    g2 = gathered.reshape(n // 16, 128)
    o0, o1 = _tc_call(g2, wp)

    def to_btS(o):
        ob = o.reshape(b, t * n_states).T
        return ob.reshape(t, n_states, b).transpose(2, 0, 1)

    o0r = to_btS(o0)
    o1r = to_btS(o1)
    return jnp.concatenate([o0r[:, :, :, None], o1r[:, :, :, None]], axis=3)
